# baseline (speedup 1.0000x reference)
import functools

import jax
import jax.numpy as jnp
from jax import lax
from jax.experimental import pallas as pl
from jax.experimental.pallas import tpu as pltpu
from jax.experimental.pallas import tpu_sc as plsc

_N_USERS = 100000
_N_ITEMS = 1000000
_EMB = 64
_U_FLAT = _N_USERS * _EMB
_I_FLAT = _N_ITEMS * _EMB

_NC = 2
_NS = 16
_NW = _NC * _NS

_U_PER_W = _U_FLAT // _NW
_I_PER_W = _I_FLAT // _NW
_CHUNK = 40000
_U_CHUNKS = _U_PER_W // _CHUNK
_I_CHUNKS = _I_PER_W // _CHUNK


def _body(user_hbm, item_hbm, out_hbm, buf, sem):
    wid = lax.axis_index("s") * _NC + lax.axis_index("c")

    ubase = wid * _U_PER_W

    def user_chunk(i, carry):
        start = ubase + i * _CHUNK
        pltpu.async_copy(user_hbm.at[pl.ds(start, _CHUNK)], buf, sem).wait()
        pltpu.sync_copy(buf, out_hbm.at[pl.ds(start, _CHUNK)])
        return carry

    lax.fori_loop(0, _U_CHUNKS, user_chunk, 0)

    ibase = wid * _I_PER_W

    def item_chunk(i, carry):
        start = ibase + i * _CHUNK
        pltpu.async_copy(item_hbm.at[pl.ds(start, _CHUNK)], buf, sem).wait()
        pltpu.sync_copy(buf, out_hbm.at[pl.ds(_U_FLAT + start, _CHUNK)])
        return carry

    lax.fori_loop(0, _I_CHUNKS, item_chunk, 0)


def kernel(user_emb, item_emb):
    mesh = plsc.VectorSubcoreMesh(core_axis_name="c", subcore_axis_name="s")
    k = functools.partial(
        pl.kernel,
        mesh=mesh,
        out_type=jax.ShapeDtypeStruct((_U_FLAT + _I_FLAT,), jnp.float32),
        scratch_types=[
            pltpu.VMEM((_CHUNK,), jnp.float32),
            pltpu.SemaphoreType.DMA,
        ],
    )(_body)
    flat = k(user_emb.reshape(_U_FLAT), item_emb.reshape(_I_FLAT))
    return flat.reshape(_N_USERS + _N_ITEMS, _EMB)


# speedup vs baseline: 1.0181x; 1.0181x over previous
import functools

import jax
import jax.numpy as jnp
from jax import lax
from jax.experimental import pallas as pl
from jax.experimental.pallas import tpu as pltpu
from jax.experimental.pallas import tpu_sc as plsc

_N_USERS = 100000
_N_ITEMS = 1000000
_EMB = 64
_U_FLAT = _N_USERS * _EMB
_I_FLAT = _N_ITEMS * _EMB

_NC = 2
_NS = 16
_NW = _NC * _NS

_U_PER_W = _U_FLAT // _NW
_I_PER_W = _I_FLAT // _NW
_CHUNK = 25000
_NBUF = 4
_U_CHUNKS = _U_PER_W // _CHUNK
_I_CHUNKS = _I_PER_W // _CHUNK


def _copy_stream(src, dst, dst_off, n_chunks, bufs, rsems, wsems):
    k_iters = n_chunks // _NBUF

    def read(g, b):
        return pltpu.async_copy(src.at[pl.ds(g * _CHUNK, _CHUNK)], bufs[b], rsems[b])

    def write(g, b):
        return pltpu.async_copy(
            bufs[b], dst.at[pl.ds(dst_off + g * _CHUNK, _CHUNK)], wsems[b]
        )

    for b in range(_NBUF):
        read(b, b)

    def body(k, carry):
        base = k * _NBUF
        for b in range(_NBUF):
            pltpu.make_async_copy(src.at[pl.ds(0, _CHUNK)], bufs[b], rsems[b]).wait()
            write(base + b, b)

        @pl.when(k < k_iters - 1)
        def _():
            for b in range(_NBUF):
                pltpu.make_async_copy(
                    bufs[b], dst.at[pl.ds(dst_off, _CHUNK)], wsems[b]
                ).wait()
                read(base + _NBUF + b, b)

        return carry

    lax.fori_loop(0, k_iters, body, 0)

    for b in range(_NBUF):
        pltpu.make_async_copy(bufs[b], dst.at[pl.ds(dst_off, _CHUNK)], wsems[b]).wait()


def _body(user_hbm, item_hbm, out_hbm, b0, b1, b2, b3, r0, r1, r2, r3, w0, w1, w2, w3):
    wid = lax.axis_index("s") * _NC + lax.axis_index("c")
    bufs = (b0, b1, b2, b3)
    rsems = (r0, r1, r2, r3)
    wsems = (w0, w1, w2, w3)

    usrc = user_hbm.at[pl.ds(wid * _U_PER_W, _U_PER_W)]
    udst_off = wid * _U_PER_W
    _copy_stream(usrc, out_hbm, udst_off, _U_CHUNKS, bufs, rsems, wsems)

    isrc = item_hbm.at[pl.ds(wid * _I_PER_W, _I_PER_W)]
    idst_off = _U_FLAT + wid * _I_PER_W
    _copy_stream(isrc, out_hbm, idst_off, _I_CHUNKS, bufs, rsems, wsems)


def kernel(user_emb, item_emb):
    mesh = plsc.VectorSubcoreMesh(core_axis_name="c", subcore_axis_name="s")
    k = functools.partial(
        pl.kernel,
        mesh=mesh,
        out_type=jax.ShapeDtypeStruct((_U_FLAT + _I_FLAT,), jnp.float32),
        scratch_types=[pltpu.VMEM((_CHUNK,), jnp.float32)] * _NBUF
        + [pltpu.SemaphoreType.DMA] * (2 * _NBUF),
    )(_body)
    flat = k(user_emb.reshape(_U_FLAT), item_emb.reshape(_I_FLAT))
    return flat.reshape(_N_USERS + _N_ITEMS, _EMB)


# speedup vs baseline: 1.3249x; 1.3014x over previous
import functools

import jax
import jax.numpy as jnp
from jax import lax
from jax.experimental import pallas as pl
from jax.experimental.pallas import tpu as pltpu
from jax.experimental.pallas import tpu_sc as plsc

_N_USERS = 100000
_N_ITEMS = 1000000
_EMB = 64

_NC = 2
_NS = 16
_NW = _NC * _NS

_NBUF = 2
_BUF_ROWS = 496

_U_ROWS = 3128
_I_ROWS = 31256
_U_SIZES = [496] * 6 + [152]
_I_SIZES = [496] * 63 + [8]


def _ring_copy(src, dst, src_base, dst_base, sizes, bufs, rsems, wsems):
    n = len(sizes)
    offs = [0]
    for s in sizes:
        offs.append(offs[-1] + s)

    def read(k, b):
        pltpu.async_copy(
            src.at[pl.ds(src_base + offs[k], sizes[k])],
            bufs[b].at[pl.ds(0, sizes[k])],
            rsems[b],
        )

    def write(k, b):
        pltpu.async_copy(
            bufs[b].at[pl.ds(0, sizes[k])],
            dst.at[pl.ds(dst_base + offs[k], sizes[k])],
            wsems[b],
        )

    def wait_read(k, b):
        pltpu.make_async_copy(
            src.at[pl.ds(src_base, sizes[k])], bufs[b].at[pl.ds(0, sizes[k])], rsems[b]
        ).wait()

    def wait_write(k, b):
        pltpu.make_async_copy(
            bufs[b].at[pl.ds(0, sizes[k])], dst.at[pl.ds(dst_base, sizes[k])], wsems[b]
        ).wait()

    for b in range(_NBUF):
        read(b, b)
    for k in range(n):
        b = k % _NBUF
        wait_read(k, b)
        write(k, b)
        nk = k + _NBUF
        if nk < n:
            wait_write(k, b)
            read(nk, b)
    for k in range(n - _NBUF, n):
        wait_write(k, k % _NBUF)


def _body(user_hbm, item_hbm, out_hbm, b0, b1, r0, r1, w0, w1):
    wid = lax.axis_index("s") * _NC + lax.axis_index("c")
    bufs = (b0, b1)
    rsems = (r0, r1)
    wsems = (w0, w1)

    ustart = pl.multiple_of(wid * (_N_USERS // _NW) // 8 * 8, 8)
    _ring_copy(user_hbm, out_hbm, ustart, ustart, _U_SIZES, bufs, rsems, wsems)

    istart = pl.multiple_of(wid * (_N_ITEMS // _NW) // 8 * 8, 8)
    _ring_copy(
        item_hbm, out_hbm, istart, _N_USERS + istart, _I_SIZES, bufs, rsems, wsems
    )


def kernel(user_emb, item_emb):
    mesh = plsc.VectorSubcoreMesh(
        core_axis_name="c", subcore_axis_name="s", num_cores=_NC
    )
    k = functools.partial(
        pl.kernel,
        mesh=mesh,
        out_type=jax.ShapeDtypeStruct((_N_USERS + _N_ITEMS, _EMB), jnp.float32),
        scratch_types=[pltpu.VMEM((_BUF_ROWS, _EMB), jnp.float32)] * _NBUF
        + [pltpu.SemaphoreType.DMA] * (2 * _NBUF),
    )(_body)
    return k(user_emb, item_emb)
